# baseline (speedup 1.0000x reference)
import functools

import jax
import jax.numpy as jnp
from jax import lax
from jax.experimental import pallas as pl
from jax.experimental.pallas import tpu as pltpu
from jax.experimental.pallas import tpu_sc as plsc

_NC = 2
_NS = 16
_NW = _NC * _NS


@functools.partial(jax.jit, static_argnums=(2, 3))
def _sc_gather(idx, table, B, D):
    b_per_w = B // _NW
    C = 64
    n_chunks = b_per_w // C

    mesh = plsc.VectorSubcoreMesh(core_axis_name="c", subcore_axis_name="s")

    @functools.partial(
        pl.kernel,
        mesh=mesh,
        out_type=jax.ShapeDtypeStruct((B, D), jnp.float32),
        scratch_types=[
            pltpu.VMEM((b_per_w,), jnp.int32),
            pltpu.VMEM((C, D), jnp.float32),
            pltpu.VMEM_SHARED((_NS, C, D), jnp.float32),
            pltpu.SemaphoreType.DMA,
            pltpu.SemaphoreType.DMA,
            pltpu.SemaphoreType.DMA,
        ],
    )
    def k(idx_hbm, table_hbm, out_hbm, idx_v, rows_v, stage, gsem, xsem, ssem):
        cid = lax.axis_index("c")
        sid = lax.axis_index("s")
        wid = sid * _NC + cid
        base = wid * b_per_w
        pltpu.sync_copy(idx_hbm.at[pl.ds(base, b_per_w)], idx_v)
        for g in range(n_chunks):
            pltpu.async_copy(
                table_hbm.at[idx_v.at[pl.ds(g * C, C)]], rows_v, gsem
            ).wait()
            pltpu.async_copy(rows_v, stage.at[sid], xsem).wait()
            pltpu.async_copy(
                stage.at[sid], out_hbm.at[pl.ds(base + g * C, C)], ssem
            ).wait()

    return k(idx, table)


def kernel(x, W_E):
    B, S = x.shape
    V, D = W_E.shape
    flat = x.reshape(B * S).astype(jnp.int32)
    out = _sc_gather(flat, W_E, B * S, D)
    return out.reshape(B, S, D)


# speedup vs baseline: 1.1699x; 1.1699x over previous
import functools

import jax
import jax.numpy as jnp
from jax import lax
from jax.experimental import pallas as pl
from jax.experimental.pallas import tpu as pltpu
from jax.experimental.pallas import tpu_sc as plsc

_NC = 2
_NS = 16
_NW = _NC * _NS


@functools.partial(jax.jit, static_argnums=(2, 3))
def _sc_gather(idx, table, B, D):
    b_per_w = B // _NW
    C = 32
    n_chunks = b_per_w // C

    mesh = plsc.VectorSubcoreMesh(core_axis_name="c", subcore_axis_name="s")

    @functools.partial(
        pl.kernel,
        mesh=mesh,
        out_type=jax.ShapeDtypeStruct((B, D), jnp.float32),
        scratch_types=[
            pltpu.VMEM((b_per_w,), jnp.int32),
            pltpu.VMEM((C, D), jnp.float32),
            pltpu.VMEM((C, D), jnp.float32),
            pltpu.VMEM_SHARED((_NS, 2, C, D), jnp.float32),
            pltpu.SemaphoreType.DMA,
            pltpu.SemaphoreType.DMA,
            pltpu.SemaphoreType.DMA,
            pltpu.SemaphoreType.DMA,
            pltpu.SemaphoreType.DMA,
            pltpu.SemaphoreType.DMA,
        ],
    )
    def k(idx_hbm, table_hbm, out_hbm, idx_v, t0, t1, stage,
          g0, g1, x0, x1, w0, w1):
        cid = lax.axis_index("c")
        sid = lax.axis_index("s")
        wid = sid * _NC + cid
        base = wid * b_per_w
        tsp = (t0, t1)
        gsem = (g0, g1)
        xsem = (x0, x1)
        wsem = (w0, w1)
        n = n_chunks
        pltpu.sync_copy(idx_hbm.at[pl.ds(base, b_per_w)], idx_v)

        def gather(g):
            return pltpu.async_copy(
                table_hbm.at[idx_v.at[pl.ds(g * C, C)]], tsp[g % 2], gsem[g % 2]
            )

        def crossbar(g):
            return pltpu.async_copy(tsp[g % 2], stage.at[sid, g % 2], xsem[g % 2])

        def write(g):
            return pltpu.async_copy(
                stage.at[sid, g % 2], out_hbm.at[pl.ds(base + g * C, C)],
                wsem[g % 2],
            )

        gops = [None] * n
        xops = [None] * n
        wops = [None] * n
        for g in range(n):
            if g >= 2:
                xops[g - 2].wait()
                wops[g - 2] = write(g - 2)
            gops[g] = gather(g)
            if g >= 1:
                gops[g - 1].wait()
                if g >= 3:
                    wops[g - 3].wait()
                xops[g - 1] = crossbar(g - 1)
        gops[n - 1].wait()
        if n >= 3:
            wops[n - 3].wait()
        xops[n - 1] = crossbar(n - 1)
        xops[n - 2].wait()
        wops[n - 2] = write(n - 2)
        xops[n - 1].wait()
        wops[n - 1] = write(n - 1)
        wops[n - 2].wait()
        wops[n - 1].wait()

    return k(idx, table)


def kernel(x, W_E):
    B, S = x.shape
    V, D = W_E.shape
    flat = x.reshape(B * S).astype(jnp.int32)
    out = _sc_gather(flat, W_E, B * S, D)
    return out.reshape(B, S, D)


# speedup vs baseline: 1.2320x; 1.0531x over previous
import functools

import jax
import jax.numpy as jnp
from jax import lax
from jax.experimental import pallas as pl
from jax.experimental.pallas import tpu as pltpu
from jax.experimental.pallas import tpu_sc as plsc

_NC = 2
_NS = 16
_NW = _NC * _NS


@jax.jit
def _sc_gather(x, table):
    Bx, S = x.shape
    V, D = table.shape
    B = Bx * S
    b_per_w = B // _NW
    C = 128
    n_chunks = b_per_w // C
    w_per_row = S // b_per_w

    mesh = plsc.VectorSubcoreMesh(core_axis_name="c", subcore_axis_name="s")

    @functools.partial(
        pl.kernel,
        mesh=mesh,
        out_type=jax.ShapeDtypeStruct((B, D), jnp.float32),
        scratch_types=[
            pltpu.VMEM((b_per_w,), jnp.int32),
            pltpu.VMEM((C, D), jnp.float32),
            pltpu.SemaphoreType.DMA,
        ],
    )
    def k(x_hbm, table_hbm, out_hbm, idx_v, rows_v, sem):
        wid = lax.axis_index("s") * _NC + lax.axis_index("c")
        base = wid * b_per_w
        r = wid // w_per_row
        col = (wid % w_per_row) * b_per_w
        pltpu.sync_copy(x_hbm.at[r, pl.ds(col, b_per_w)], idx_v)
        for g in range(n_chunks):
            pltpu.async_copy(
                table_hbm.at[idx_v.at[pl.ds(g * C, C)]], rows_v, sem
            ).wait()
            pltpu.sync_copy(rows_v, out_hbm.at[pl.ds(base + g * C, C)])

    return k(x, table)


def kernel(x, W_E):
    B, S = x.shape
    V, D = W_E.shape
    out = _sc_gather(x.astype(jnp.int32), W_E)
    return out.reshape(B, S, D)
